# baseline (device time: 113171 ns/iter reference)
import jax
import jax.numpy as jnp
from jax import lax
from jax.experimental import pallas as pl
from jax.experimental.pallas import tpu as pltpu

CHUNK_ROWS = [160] * 10 + [64] * 7


def kernel(x, pi):
    _, m, n = x.shape
    half = m // 2
    assert sum(CHUNK_ROWS) == half
    n_chunks = len(CHUNK_ROWS)
    starts = [sum(CHUNK_ROWS[:c]) for c in range(n_chunks)]
    max_rows = max(CHUNK_ROWS)

    def body(pi_ref, x_ref, out_ref, load_buf, send_buf,
             load_sems, xsend_sems, xrecv_sems, fsend_sems, frecv_sems):
        my_x = lax.axis_index("x")
        my_y = lax.axis_index("y")
        dst_x = pi_ref[my_x]
        src_x = jnp.where(pi_ref[0] == my_x, 0, 1)
        half_base = my_y * half

        barrier_sem = pltpu.get_barrier_semaphore()
        pl.semaphore_signal(
            barrier_sem,
            inc=1,
            device_id=(src_x, my_y),
            device_id_type=pl.DeviceIdType.MESH,
        )
        pl.semaphore_signal(
            barrier_sem,
            inc=1,
            device_id=(my_x, 1 - my_y),
            device_id_type=pl.DeviceIdType.MESH,
        )

        def load(c, slot):
            rc = CHUNK_ROWS[c]
            return pltpu.make_async_copy(
                x_ref.at[0, pl.ds(half_base + starts[c], rc), :],
                load_buf.at[slot, pl.ds(0, rc), :],
                load_sems.at[slot],
            )

        load(0, 0).start()
        x_rdmas = []
        for c in range(n_chunks):
            rc = CHUNK_ROWS[c]
            slot = c % 2
            if c + 1 < n_chunks:
                load(c + 1, (c + 1) % 2).start()
            load(c, slot).wait()
            send_buf[pl.ds(starts[c], rc), :] = load_buf[
                slot, pl.ds(0, rc), :
            ].astype(jnp.bfloat16)
            if c == 0:
                pl.semaphore_wait(barrier_sem, 2)
            rdma = pltpu.make_async_remote_copy(
                src_ref=send_buf.at[pl.ds(starts[c], rc), :],
                dst_ref=out_ref.at[0, pl.ds(half_base + starts[c], rc), :],
                send_sem=xsend_sems.at[c],
                recv_sem=xrecv_sems.at[c],
                device_id=(dst_x, my_y),
                device_id_type=pl.DeviceIdType.MESH,
            )
            rdma.start()
            x_rdmas.append(rdma)

        fwds = []
        for c in range(n_chunks):
            rc = CHUNK_ROWS[c]
            x_rdmas[c].wait_recv()
            fwd = pltpu.make_async_remote_copy(
                src_ref=out_ref.at[0, pl.ds(half_base + starts[c], rc), :],
                dst_ref=out_ref.at[0, pl.ds(half_base + starts[c], rc), :],
                send_sem=fsend_sems.at[c],
                recv_sem=frecv_sems.at[c],
                device_id=(my_x, 1 - my_y),
                device_id_type=pl.DeviceIdType.MESH,
            )
            fwd.start()
            fwds.append(fwd)

        for c in range(n_chunks):
            x_rdmas[c].wait_send()
            fwds[c].wait_send()
            fwds[c].wait_recv()

    return pl.pallas_call(
        body,
        out_shape=jax.ShapeDtypeStruct((1, m, n), jnp.bfloat16),
        in_specs=[
            pl.BlockSpec(memory_space=pltpu.SMEM),
            pl.BlockSpec(memory_space=pl.ANY),
        ],
        out_specs=pl.BlockSpec(memory_space=pl.ANY),
        scratch_shapes=[
            pltpu.VMEM((2, max_rows, n), jnp.float32),
            pltpu.VMEM((half, n), jnp.bfloat16),
            pltpu.SemaphoreType.DMA((2,)),
            pltpu.SemaphoreType.DMA((n_chunks,)),
            pltpu.SemaphoreType.DMA((n_chunks,)),
            pltpu.SemaphoreType.DMA((n_chunks,)),
            pltpu.SemaphoreType.DMA((n_chunks,)),
        ],
        compiler_params=pltpu.CompilerParams(collective_id=0),
    )(pi, x)


# device time: 112055 ns/iter; 1.0100x vs baseline; 1.0100x over previous
import jax
import jax.numpy as jnp
from jax import lax
from jax.experimental import pallas as pl
from jax.experimental.pallas import tpu as pltpu

CHUNK_ROWS = [128] * 16


def kernel(x, pi):
    _, m, n = x.shape
    half = m // 2
    assert sum(CHUNK_ROWS) == half
    n_chunks = len(CHUNK_ROWS)
    starts = [sum(CHUNK_ROWS[:c]) for c in range(n_chunks)]
    max_rows = max(CHUNK_ROWS)

    def body(pi_ref, x_ref, out_ref, load_buf, send_buf,
             load_sems, xsend_sems, xrecv_sems, fsend_sems, frecv_sems):
        my_x = lax.axis_index("x")
        my_y = lax.axis_index("y")
        dst_x = pi_ref[my_x]
        src_x = jnp.where(pi_ref[0] == my_x, 0, 1)
        half_base = my_y * half

        barrier_sem = pltpu.get_barrier_semaphore()
        pl.semaphore_signal(
            barrier_sem,
            inc=1,
            device_id=(src_x, my_y),
            device_id_type=pl.DeviceIdType.MESH,
        )
        pl.semaphore_signal(
            barrier_sem,
            inc=1,
            device_id=(my_x, 1 - my_y),
            device_id_type=pl.DeviceIdType.MESH,
        )

        def load(c, slot):
            rc = CHUNK_ROWS[c]
            return pltpu.make_async_copy(
                x_ref.at[0, pl.ds(half_base + starts[c], rc), :],
                load_buf.at[slot, pl.ds(0, rc), :],
                load_sems.at[slot],
            )

        load(0, 0).start()
        x_rdmas = []
        for c in range(n_chunks):
            rc = CHUNK_ROWS[c]
            slot = c % 2
            if c + 1 < n_chunks:
                load(c + 1, (c + 1) % 2).start()
            load(c, slot).wait()
            send_buf[pl.ds(starts[c], rc), :] = load_buf[
                slot, pl.ds(0, rc), :
            ].astype(jnp.bfloat16)
            if c == 0:
                pl.semaphore_wait(barrier_sem, 2)
            rdma = pltpu.make_async_remote_copy(
                src_ref=send_buf.at[pl.ds(starts[c], rc), :],
                dst_ref=out_ref.at[0, pl.ds(half_base + starts[c], rc), :],
                send_sem=xsend_sems.at[c],
                recv_sem=xrecv_sems.at[c],
                device_id=(dst_x, my_y),
                device_id_type=pl.DeviceIdType.MESH,
            )
            rdma.start()
            x_rdmas.append(rdma)

        fwds = []
        for c in range(n_chunks):
            rc = CHUNK_ROWS[c]
            x_rdmas[c].wait_recv()
            fwd = pltpu.make_async_remote_copy(
                src_ref=out_ref.at[0, pl.ds(half_base + starts[c], rc), :],
                dst_ref=out_ref.at[0, pl.ds(half_base + starts[c], rc), :],
                send_sem=fsend_sems.at[c],
                recv_sem=frecv_sems.at[c],
                device_id=(my_x, 1 - my_y),
                device_id_type=pl.DeviceIdType.MESH,
            )
            fwd.start()
            fwds.append(fwd)

        for c in range(n_chunks):
            x_rdmas[c].wait_send()
            fwds[c].wait_send()
            fwds[c].wait_recv()

    return pl.pallas_call(
        body,
        out_shape=jax.ShapeDtypeStruct((1, m, n), jnp.bfloat16),
        in_specs=[
            pl.BlockSpec(memory_space=pltpu.SMEM),
            pl.BlockSpec(memory_space=pl.ANY),
        ],
        out_specs=pl.BlockSpec(memory_space=pl.ANY),
        scratch_shapes=[
            pltpu.VMEM((2, max_rows, n), jnp.float32),
            pltpu.VMEM((half, n), jnp.bfloat16),
            pltpu.SemaphoreType.DMA((2,)),
            pltpu.SemaphoreType.DMA((n_chunks,)),
            pltpu.SemaphoreType.DMA((n_chunks,)),
            pltpu.SemaphoreType.DMA((n_chunks,)),
            pltpu.SemaphoreType.DMA((n_chunks,)),
        ],
        compiler_params=pltpu.CompilerParams(collective_id=0),
    )(pi, x)
